# Initial kernel scaffold; baseline (speedup 1.0000x reference)
import jax
import jax.numpy as jnp
from jax.experimental import pallas as pl


def kernel(x, edge_index, Wl1, Wr1, b1, gamma, beta, Wl2, Wr2, b2):
    raise NotImplementedError("write your pallas kernel here")


# speedup vs baseline: 6.3386x; 6.3386x over previous
import functools

import jax
import jax.numpy as jnp
from jax import lax
from jax.experimental import pallas as pl
from jax.experimental.pallas import tpu as pltpu
from jax.experimental.pallas import tpu_sc as plsc

N = 10000
E = 320000
DIN = 128
DH = 512
DOUT = 128

NC = 2
NS = 16
NW = NC * NS
CHUNK = 128
CH = -(-E // (NW * CHUNK))
E_PAD = NW * CH * CHUNK
N_PAD = 10240
ROWS_PER_SUB = N_PAD // NS
CNTW = 16

BLK = 2000


@functools.lru_cache(maxsize=None)
def _make_sc_agg(with_counts: bool):
    mesh = plsc.VectorSubcoreMesh(core_axis_name="c", subcore_axis_name="s",
                                  num_cores=NC, num_subcores=NS)

    out_type = [jax.ShapeDtypeStruct((NC, N_PAD, DIN), jnp.float32)]
    scratch = [
        pltpu.VMEM((CHUNK,), jnp.int32),
        pltpu.VMEM((CHUNK,), jnp.int32),
        pltpu.VMEM((CHUNK, DIN), jnp.float32),
        pltpu.VMEM_SHARED((N_PAD, DIN), jnp.float32),
        pltpu.SemaphoreType.DMA,
    ]
    if with_counts:
        out_type.append(jax.ShapeDtypeStruct((NC, N_PAD, CNTW), jnp.float32))
        scratch += [
            pltpu.VMEM((CHUNK, CNTW), jnp.float32),
            pltpu.VMEM_SHARED((N_PAD, CNTW), jnp.float32),
        ]

    @functools.partial(
        pl.kernel,
        out_type=out_type,
        mesh=mesh,
        scratch_types=scratch,
        compiler_params=pltpu.CompilerParams(use_tc_tiling_on_sc=False),
    )
    def sc_agg(*refs):
        if with_counts:
            (table, src1, dst1, zrow, zcnt, ones,
             acc_out, cnt_out,
             sidx, didx, rows, acc_sh, sem, ones_v, cnt_sh) = refs
        else:
            (table, src1, dst1, zrow,
             acc_out,
             sidx, didx, rows, acc_sh, sem) = refs

        c = lax.axis_index("c")
        s = lax.axis_index("s")
        w = c * NS + s

        base = s * ROWS_PER_SUB
        pltpu.sync_copy(zrow.at[pl.ds(base, ROWS_PER_SUB)],
                        acc_sh.at[pl.ds(base, ROWS_PER_SUB)])
        if with_counts:
            pltpu.sync_copy(zcnt.at[pl.ds(base, ROWS_PER_SUB)],
                            cnt_sh.at[pl.ds(base, ROWS_PER_SUB)])
            pltpu.sync_copy(ones, ones_v)
        plsc.subcore_barrier()

        def body(j, carry):
            off = (w * CH + j) * CHUNK
            pltpu.sync_copy(src1.at[pl.ds(off, CHUNK)], sidx)
            pltpu.sync_copy(dst1.at[pl.ds(off, CHUNK)], didx)
            pltpu.async_copy(table.at[sidx], rows, sem).wait()
            pltpu.sync_copy(rows, acc_sh.at[didx], add=True)
            if with_counts:
                pltpu.sync_copy(ones_v, cnt_sh.at[didx], add=True)
            return carry

        lax.fori_loop(0, CH, body, 0)
        plsc.subcore_barrier()

        pltpu.sync_copy(acc_sh.at[pl.ds(base, ROWS_PER_SUB)],
                        acc_out.at[c, pl.ds(base, ROWS_PER_SUB)])
        if with_counts:
            pltpu.sync_copy(cnt_sh.at[pl.ds(base, ROWS_PER_SUB)],
                            cnt_out.at[c, pl.ds(base, ROWS_PER_SUB)])

    return sc_agg


def _k1_body(a0, a1, c0, c1, xr, wl, wr, b, h_out, stats):
    cnt = jnp.maximum(c0[:, :1] + c1[:, :1], 1.0)
    agg = (a0[...] + a1[...]) / cnt
    h = (lax.dot_general(agg, wl[...], (((1,), (1,)), ((), ())),
                         preferred_element_type=jnp.float32)
         + lax.dot_general(xr[...], wr[...], (((1,), (1,)), ((), ())),
                           preferred_element_type=jnp.float32)
         + b[...])
    h_out[...] = h

    @pl.when(pl.program_id(0) == 0)
    def _():
        stats[...] = jnp.zeros_like(stats)

    s1 = jnp.sum(h, axis=0, keepdims=True)
    s2 = jnp.sum(h * h, axis=0, keepdims=True)
    upd = jnp.concatenate([s1, s2, jnp.zeros((6, DH), jnp.float32)], axis=0)
    stats[...] = stats[...] + upd


def _k1_call(acc, c0, c1, x, Wl1, Wr1, b1):
    a = acc[:, :N]
    grid = (N // BLK,)
    return pl.pallas_call(
        _k1_body,
        grid=grid,
        in_specs=[
            pl.BlockSpec((BLK, DIN), lambda i: (i, 0)),
            pl.BlockSpec((BLK, DIN), lambda i: (i, 0)),
            pl.BlockSpec((BLK, CNTW), lambda i: (i, 0)),
            pl.BlockSpec((BLK, CNTW), lambda i: (i, 0)),
            pl.BlockSpec((BLK, DIN), lambda i: (i, 0)),
            pl.BlockSpec((DH, DIN), lambda i: (0, 0)),
            pl.BlockSpec((DH, DIN), lambda i: (0, 0)),
            pl.BlockSpec((1, DH), lambda i: (0, 0)),
        ],
        out_specs=[
            pl.BlockSpec((BLK, DH), lambda i: (i, 0)),
            pl.BlockSpec((8, DH), lambda i: (0, 0)),
        ],
        out_shape=[
            jax.ShapeDtypeStruct((N, DH), jnp.float32),
            jax.ShapeDtypeStruct((8, DH), jnp.float32),
        ],
    )(a[0], a[1], c0, c1, x, Wl1, Wr1, b1[None])


def _k2_body(h, stats, g, bt, wl2, wr2, b2, p_out, r_out):
    mu = stats[0:1] / N
    var = stats[1:2] / N - mu * mu
    rstd = lax.rsqrt(var + 1e-5)
    scale = g[...] * rstd
    shift = bt[...] - mu * scale
    hn = jnp.maximum(h[...] * scale + shift, 0.0)
    p_out[...] = lax.dot_general(hn, wl2[...], (((1,), (1,)), ((), ())),
                                 preferred_element_type=jnp.float32)
    r_out[...] = lax.dot_general(hn, wr2[...], (((1,), (1,)), ((), ())),
                                 preferred_element_type=jnp.float32) + b2[...]


def _k2_call(h, stats, gamma, beta, Wl2, Wr2, b2):
    grid = (N // BLK,)
    return pl.pallas_call(
        _k2_body,
        grid=grid,
        in_specs=[
            pl.BlockSpec((BLK, DH), lambda i: (i, 0)),
            pl.BlockSpec((8, DH), lambda i: (0, 0)),
            pl.BlockSpec((1, DH), lambda i: (0, 0)),
            pl.BlockSpec((1, DH), lambda i: (0, 0)),
            pl.BlockSpec((DOUT, DH), lambda i: (0, 0)),
            pl.BlockSpec((DOUT, DH), lambda i: (0, 0)),
            pl.BlockSpec((1, DOUT), lambda i: (0, 0)),
        ],
        out_specs=[
            pl.BlockSpec((BLK, DOUT), lambda i: (i, 0)),
            pl.BlockSpec((BLK, DOUT), lambda i: (i, 0)),
        ],
        out_shape=[
            jax.ShapeDtypeStruct((N, DOUT), jnp.float32),
            jax.ShapeDtypeStruct((N, DOUT), jnp.float32),
        ],
    )(h, stats, gamma[None], beta[None], Wl2, Wr2, b2[None])


def _k3_body(a0, a1, c0, c1, r2, out):
    cnt = jnp.maximum(c0[:, :1] + c1[:, :1], 1.0)
    out[...] = (a0[...] + a1[...]) / cnt + r2[...]


def _k3_call(acc2, c0, c1, r2):
    a = acc2[:, :N]
    grid = (N // BLK,)
    return pl.pallas_call(
        _k3_body,
        grid=grid,
        in_specs=[
            pl.BlockSpec((BLK, DOUT), lambda i: (i, 0)),
            pl.BlockSpec((BLK, DOUT), lambda i: (i, 0)),
            pl.BlockSpec((BLK, CNTW), lambda i: (i, 0)),
            pl.BlockSpec((BLK, CNTW), lambda i: (i, 0)),
            pl.BlockSpec((BLK, DOUT), lambda i: (i, 0)),
        ],
        out_specs=pl.BlockSpec((BLK, DOUT), lambda i: (i, 0)),
        out_shape=jax.ShapeDtypeStruct((N, DOUT), jnp.float32),
    )(a[0], a[1], c0, c1, r2)


def kernel(x, edge_index, Wl1, Wr1, b1, gamma, beta, Wl2, Wr2, b2):
    src = edge_index[0]
    dst = edge_index[1]
    pad = E_PAD - E
    src1 = jnp.concatenate([src, jnp.zeros((pad,), jnp.int32)])
    dst1 = jnp.concatenate([dst, jnp.full((pad,), N, jnp.int32)])

    zrow = jnp.zeros((N_PAD, DIN), jnp.float32)
    zcnt = jnp.zeros((N_PAD, CNTW), jnp.float32)
    ones = jnp.ones((CHUNK, CNTW), jnp.float32)

    acc1, cnt = _make_sc_agg(True)(x, src1, dst1, zrow, zcnt, ones)
    c0 = cnt[0, :N]
    c1 = cnt[1, :N]
    h_pre, stats = _k1_call(acc1, c0, c1, x, Wl1, Wr1, b1)
    p2, r2 = _k2_call(h_pre, stats, gamma, beta, Wl2, Wr2, b2)
    (acc2,) = _make_sc_agg(False)(p2, src1, dst1, zrow)
    return _k3_call(acc2, c0, c1, r2)
